# baseline (speedup 1.0000x reference)
import functools

import jax
import jax.numpy as jnp
from jax.experimental import pallas as pl
from jax.experimental.pallas import tpu as pltpu


def _fused_kernel(x_ref, wih_ref, b_ref, whh_ref, bn_ref, len_ref,
                  hfin_ref, out_ref, h_ref, gi_ref, *, ts, nb):
    i = pl.program_id(0)

    @pl.when(i == 0)
    def _():
        h_ref[...] = jnp.zeros_like(h_ref)

    bsz = x_ref.shape[0]
    dim = x_ref.shape[2]
    hdim = h_ref.shape[1]
    gdim = 3 * hdim

    base = (nb - 1 - i) * ts
    lensv = len_ref[...]

    xb = x_ref[...].reshape(bsz * ts, dim).astype(jnp.bfloat16)
    gi = (
        jnp.dot(xb, wih_ref[...], preferred_element_type=jnp.float32)
        + b_ref[...]
    ).reshape(bsz, ts, gdim)
    gi_ref[...] = gi

    wrz = whh_ref[:, :2 * hdim]
    wn = whh_ref[:, 2 * hdim:]
    bn = bn_ref[...]
    h = h_ref[...]
    for k in range(ts - 1, -1, -1):
        g = gi_ref[:, k, :]
        hb = h.astype(jnp.bfloat16)
        ghrz = jnp.dot(hb, wrz, preferred_element_type=jnp.float32)
        ghn = jnp.dot(hb, wn, preferred_element_type=jnp.float32)
        r = jax.nn.sigmoid(g[:, :hdim] + ghrz[:, :hdim])
        z = jax.nn.sigmoid(g[:, hdim:2 * hdim] + ghrz[:, hdim:])
        n = jnp.tanh(g[:, 2 * hdim:] + r * (ghn + bn))
        h_new = (1.0 - z) * n + z * h
        valid = lensv > (base + k)
        h = jnp.where(valid, h_new, 0.0)
        out_ref[:, k, :] = h
    h_ref[...] = h
    hfin_ref[...] = h


def kernel(inp, lens, W_ih, W_hh, b_ih, b_hh):
    B, S, D = inp.shape
    H = W_hh.shape[1]
    G = 3 * H

    W_ihT = W_ih.T.astype(jnp.bfloat16)
    W_hhT = W_hh.T.astype(jnp.bfloat16)
    bias = jnp.concatenate([b_ih[:2 * H] + b_hh[:2 * H], b_ih[2 * H:]])
    bias = bias.reshape(1, G)
    b_hhn = b_hh[2 * H:].reshape(1, H)
    lens2 = lens.astype(jnp.int32).reshape(B, 1)

    TS = 128
    NB = S // TS
    hfin, out = pl.pallas_call(
        functools.partial(_fused_kernel, ts=TS, nb=NB),
        grid=(NB,),
        in_specs=[
            pl.BlockSpec((B, TS, D), lambda i: (0, NB - 1 - i, 0)),
            pl.BlockSpec((D, G), lambda i: (0, 0)),
            pl.BlockSpec((1, G), lambda i: (0, 0)),
            pl.BlockSpec((H, G), lambda i: (0, 0)),
            pl.BlockSpec((1, H), lambda i: (0, 0)),
            pl.BlockSpec((B, 1), lambda i: (0, 0)),
        ],
        out_specs=[
            pl.BlockSpec((B, H), lambda i: (0, 0)),
            pl.BlockSpec((B, TS, H), lambda i: (0, NB - 1 - i, 0)),
        ],
        out_shape=[
            jax.ShapeDtypeStruct((B, H), jnp.float32),
            jax.ShapeDtypeStruct((B, S, H), jnp.float32),
        ],
        scratch_shapes=[
            pltpu.VMEM((B, H), jnp.float32),
            pltpu.VMEM((B, TS, G), jnp.float32),
        ],
    )(inp, W_ihT, bias, W_hhT, b_hhn, lens2)

    return (hfin, out)


# speedup vs baseline: 1.0047x; 1.0047x over previous
import functools

import jax
import jax.numpy as jnp
from jax.experimental import pallas as pl
from jax.experimental.pallas import tpu as pltpu


def _fused_kernel(x_ref, wih_ref, b_ref, whh_ref, bn_ref, len_ref,
                  hfin_ref, out_ref, h_ref, gi_ref, *, ts, nb):
    i = pl.program_id(0)

    @pl.when(i == 0)
    def _():
        h_ref[...] = jnp.zeros_like(h_ref)

    bsz = x_ref.shape[0]
    dim = x_ref.shape[2]
    hdim = h_ref.shape[1]
    gdim = 3 * hdim

    base = (nb - 1 - i) * ts
    lensv = len_ref[...]

    xb = x_ref[...].reshape(bsz * ts, dim).astype(jnp.bfloat16)
    gi = (
        jnp.dot(xb, wih_ref[...], preferred_element_type=jnp.float32)
        + b_ref[...]
    ).reshape(bsz, ts, gdim)
    gi_ref[...] = gi

    wrz = whh_ref[:, :2 * hdim]
    wn = whh_ref[:, 2 * hdim:]
    bn = bn_ref[...]
    h = h_ref[...]
    for k in range(ts - 1, -1, -1):
        g = gi_ref[:, k, :]
        hb = h.astype(jnp.bfloat16)
        ghrz = jnp.dot(hb, wrz, preferred_element_type=jnp.float32)
        ghn = jnp.dot(hb, wn, preferred_element_type=jnp.float32)
        r = jax.nn.sigmoid(g[:, :hdim] + ghrz[:, :hdim])
        z = jax.nn.sigmoid(g[:, hdim:2 * hdim] + ghrz[:, hdim:])
        n = jnp.tanh(g[:, 2 * hdim:] + r * (ghn + bn))
        h_new = (1.0 - z) * n + z * h
        valid = lensv > (base + k)
        h = jnp.where(valid, h_new, 0.0)
        out_ref[:, k, :] = h
    h_ref[...] = h
    hfin_ref[...] = h


def kernel(inp, lens, W_ih, W_hh, b_ih, b_hh):
    B, S, D = inp.shape
    H = W_hh.shape[1]
    G = 3 * H

    W_ihT = W_ih.T.astype(jnp.bfloat16)
    W_hhT = W_hh.T.astype(jnp.bfloat16)
    bias = jnp.concatenate([b_ih[:2 * H] + b_hh[:2 * H], b_ih[2 * H:]])
    bias = bias.reshape(1, G)
    b_hhn = b_hh[2 * H:].reshape(1, H)
    lens2 = lens.astype(jnp.int32).reshape(B, 1)

    TS = 64
    NB = S // TS
    hfin, out = pl.pallas_call(
        functools.partial(_fused_kernel, ts=TS, nb=NB),
        grid=(NB,),
        in_specs=[
            pl.BlockSpec((B, TS, D), lambda i: (0, NB - 1 - i, 0)),
            pl.BlockSpec((D, G), lambda i: (0, 0)),
            pl.BlockSpec((1, G), lambda i: (0, 0)),
            pl.BlockSpec((H, G), lambda i: (0, 0)),
            pl.BlockSpec((1, H), lambda i: (0, 0)),
            pl.BlockSpec((B, 1), lambda i: (0, 0)),
        ],
        out_specs=[
            pl.BlockSpec((B, H), lambda i: (0, 0)),
            pl.BlockSpec((B, TS, H), lambda i: (0, NB - 1 - i, 0)),
        ],
        out_shape=[
            jax.ShapeDtypeStruct((B, H), jnp.float32),
            jax.ShapeDtypeStruct((B, S, H), jnp.float32),
        ],
        scratch_shapes=[
            pltpu.VMEM((B, H), jnp.float32),
            pltpu.VMEM((B, TS, G), jnp.float32),
        ],
    )(inp, W_ihT, bias, W_hhT, b_hhn, lens2)

    return (hfin, out)
